# baseline (speedup 1.0000x reference)
import jax
import jax.numpy as jnp
from jax import lax
from jax.experimental import pallas as pl
from jax.experimental.pallas import tpu as pltpu
from jax.experimental.pallas import tpu_sc as plsc

NUM_CORES = 2
NUM_SUBCORES = 16
LANES = 16
NUM_WORKERS = NUM_CORES * NUM_SUBCORES

NUM_CUSTOMERS = 1000000
NUM_PRODUCTS = 100000
BATCH = 16384
EMBED_DIM = 64
B_PER_W = BATCH // NUM_WORKERS
GROUPS = B_PER_W // LANES
ROWS = EMBED_DIM * LANES // 128


def _build(idx_v, g, cix, pix):
    cvec = idx_v[0][pl.ds(g * LANES, LANES)]
    pvec = idx_v[1][pl.ds(g * LANES, LANES)]
    for c in range(EMBED_DIM):
        sl = pl.ds((c % 8) * LANES, LANES)
        cix[c // 8, sl] = cvec + c * NUM_CUSTOMERS
        pix[c // 8, sl] = pvec + c * NUM_PRODUCTS


def _fire(ctab1, ptab1, cix, pix, cbuf, pbuf, csem, psem):
    for j in range(ROWS):
        pltpu.async_copy(ctab1.at[cix.at[j]], cbuf.at[j], csem)
        pltpu.async_copy(ptab1.at[pix.at[j]], pbuf.at[j], psem)


def _drain(ctab1, ptab1, cix, pix, cbuf, pbuf, csem, psem):
    for j in range(ROWS):
        pltpu.make_async_copy(ctab1.at[cix.at[j]], cbuf.at[j], csem).wait()
        pltpu.make_async_copy(ptab1.at[pix.at[j]], pbuf.at[j], psem).wait()


def _dot(cbuf, pbuf):
    acc = jnp.zeros((LANES,), jnp.float32)
    for c in range(EMBED_DIM):
        sl = pl.ds((c % 8) * LANES, LANES)
        acc = acc + cbuf[c // 8, sl] * pbuf[c // 8, sl]
    return acc


def _sc_body(cidx_hbm, pidx_hbm, ctab1, ptab1, out_hbm,
             cidx_v, pidx_v, out_v,
             cbuf_a, pbuf_a, cbuf_b, pbuf_b,
             cix_a, pix_a, cix_b, pix_b,
             csem_a, psem_a, csem_b, psem_b):
    wid = lax.axis_index("s") * NUM_CORES + lax.axis_index("c")
    base = wid * B_PER_W

    pltpu.sync_copy(cidx_hbm.at[pl.ds(base, B_PER_W)], cidx_v)
    pltpu.sync_copy(pidx_hbm.at[pl.ds(base, B_PER_W)], pidx_v)
    idx_v = (cidx_v, pidx_v)

    def pair_body(t, carry):
        g0 = 2 * t
        g1 = 2 * t + 1
        _build(idx_v, g0, cix_a, pix_a)
        _fire(ctab1, ptab1, cix_a, pix_a, cbuf_a, pbuf_a, csem_a, psem_a)
        _build(idx_v, g1, cix_b, pix_b)
        _fire(ctab1, ptab1, cix_b, pix_b, cbuf_b, pbuf_b, csem_b, psem_b)
        _drain(ctab1, ptab1, cix_a, pix_a, cbuf_a, pbuf_a, csem_a, psem_a)
        out_v[pl.ds(g0 * LANES, LANES)] = _dot(cbuf_a, pbuf_a)
        _drain(ctab1, ptab1, cix_b, pix_b, cbuf_b, pbuf_b, csem_b, psem_b)
        out_v[pl.ds(g1 * LANES, LANES)] = _dot(cbuf_b, pbuf_b)
        return carry

    lax.fori_loop(0, GROUPS // 2, pair_body, 0)

    pltpu.sync_copy(out_v, out_hbm.at[pl.ds(base, B_PER_W)])


@jax.jit
def _run(customer, product, customer_table, product_table):
    mesh = plsc.VectorSubcoreMesh(core_axis_name="c", subcore_axis_name="s",
                                  num_cores=NUM_CORES,
                                  num_subcores=NUM_SUBCORES)
    ctab1 = customer_table.T.reshape(-1)
    ptab1 = product_table.T.reshape(-1)
    return pl.kernel(
        _sc_body,
        out_type=jax.ShapeDtypeStruct((BATCH,), jnp.float32),
        mesh=mesh,
        compiler_params=pltpu.CompilerParams(needs_layout_passes=False,
                                             use_tc_tiling_on_sc=False),
        scratch_types=[
            pltpu.VMEM((B_PER_W,), jnp.int32),
            pltpu.VMEM((B_PER_W,), jnp.int32),
            pltpu.VMEM((B_PER_W,), jnp.float32),
            pltpu.VMEM((ROWS, 128), jnp.float32),
            pltpu.VMEM((ROWS, 128), jnp.float32),
            pltpu.VMEM((ROWS, 128), jnp.float32),
            pltpu.VMEM((ROWS, 128), jnp.float32),
            pltpu.VMEM((ROWS, 128), jnp.int32),
            pltpu.VMEM((ROWS, 128), jnp.int32),
            pltpu.VMEM((ROWS, 128), jnp.int32),
            pltpu.VMEM((ROWS, 128), jnp.int32),
            pltpu.SemaphoreType.DMA,
            pltpu.SemaphoreType.DMA,
            pltpu.SemaphoreType.DMA,
            pltpu.SemaphoreType.DMA,
        ],
    )(customer, product, ctab1, ptab1)


def kernel(customer, product, customer_table, product_table):
    return _run(customer, product, customer_table, product_table)


# speedup vs baseline: 7.3835x; 7.3835x over previous
import jax
import jax.numpy as jnp
from jax import lax
from jax.experimental import pallas as pl
from jax.experimental.pallas import tpu as pltpu
from jax.experimental.pallas import tpu_sc as plsc

NUM_CORES = 2
NUM_SUBCORES = 16
LANES = 16
NUM_WORKERS = NUM_CORES * NUM_SUBCORES

NUM_CUSTOMERS = 1000000
NUM_PRODUCTS = 100000
BATCH = 16384
EMBED_DIM = 64
B_PER_W = BATCH // NUM_WORKERS
GROUPS = B_PER_W // LANES
OUT_STRIDE = 2 * B_PER_W


def _idx_vec(idx_s, g):
    e = g * LANES + lax.iota(jnp.int32, LANES)
    return plsc.load_gather(idx_s, [lax.shift_right_logical(e, 7),
                                    jnp.bitwise_and(e, 127)])


def _gather_pair(tab2, idx_s, g, buf, sem):
    rows = lax.shift_right_logical(_idx_vec(idx_s, g), 1)
    return pltpu.async_copy(tab2.at[rows], buf, sem)


def _wait_pair(tab2, idx_s, g, buf, sem):
    rows = lax.shift_right_logical(_idx_vec(idx_s, g), 1)
    pltpu.make_async_copy(tab2.at[rows], buf, sem).wait()


def _dot(cidx_s, pidx_s, g, cbuf, pbuf, out_v):
    lane = lax.iota(jnp.int32, LANES)
    chalf = jnp.bitwise_and(_idx_vec(cidx_s, g), 1) * EMBED_DIM
    phalf = jnp.bitwise_and(_idx_vec(pidx_s, g), 1) * EMBED_DIM
    acc = jnp.zeros((LANES,), jnp.float32)
    for d in range(EMBED_DIM):
        cv = plsc.load_gather(cbuf, [lane, chalf + d])
        pv = plsc.load_gather(pbuf, [lane, phalf + d])
        acc = acc + cv * pv
    plsc.store_scatter(out_v, [g * LANES + lane], acc)


def _sc_body(cidx_hbm, pidx_hbm, ctab2, ptab2, out_hbm,
             cidx_s, pidx_s, out_v,
             cbuf_a, pbuf_a, cbuf_b, pbuf_b,
             csem_a, psem_a, csem_b, psem_b):
    wid = lax.axis_index("s") * NUM_CORES + lax.axis_index("c")

    pltpu.sync_copy(cidx_hbm.at[wid], cidx_s)
    pltpu.sync_copy(pidx_hbm.at[wid], pidx_s)

    def pair_body(t, carry):
        g0 = 2 * t
        g1 = 2 * t + 1
        _gather_pair(ctab2, cidx_s, g0, cbuf_a, csem_a)
        _gather_pair(ptab2, pidx_s, g0, pbuf_a, psem_a)
        _gather_pair(ctab2, cidx_s, g1, cbuf_b, csem_b)
        _gather_pair(ptab2, pidx_s, g1, pbuf_b, psem_b)
        _wait_pair(ctab2, cidx_s, g0, cbuf_a, csem_a)
        _wait_pair(ptab2, pidx_s, g0, pbuf_a, psem_a)
        _dot(cidx_s, pidx_s, g0, cbuf_a, pbuf_a, out_v)
        _wait_pair(ctab2, cidx_s, g1, cbuf_b, csem_b)
        _wait_pair(ptab2, pidx_s, g1, pbuf_b, psem_b)
        _dot(cidx_s, pidx_s, g1, cbuf_b, pbuf_b, out_v)
        return carry

    lax.fori_loop(0, GROUPS // 2, pair_body, 0)

    pltpu.sync_copy(out_v, out_hbm.at[pl.ds(wid * OUT_STRIDE, OUT_STRIDE)])


@jax.jit
def _run(customer, product, customer_table, product_table):
    mesh = plsc.VectorSubcoreMesh(core_axis_name="c", subcore_axis_name="s",
                                  num_cores=NUM_CORES,
                                  num_subcores=NUM_SUBCORES)
    cidx = jnp.pad(customer.reshape(NUM_WORKERS, 4, 128),
                   ((0, 0), (0, 4), (0, 0)))
    pidx = jnp.pad(product.reshape(NUM_WORKERS, 4, 128),
                   ((0, 0), (0, 4), (0, 0)))
    ctab2 = customer_table.reshape(NUM_CUSTOMERS // 2, 2 * EMBED_DIM)
    ptab2 = product_table.reshape(NUM_PRODUCTS // 2, 2 * EMBED_DIM)
    padded = pl.kernel(
        _sc_body,
        out_type=jax.ShapeDtypeStruct((NUM_WORKERS * OUT_STRIDE,),
                                      jnp.float32),
        mesh=mesh,
        compiler_params=pltpu.CompilerParams(needs_layout_passes=False),
        scratch_types=[
            pltpu.VMEM((8, 128), jnp.int32),
            pltpu.VMEM((8, 128), jnp.int32),
            pltpu.VMEM((OUT_STRIDE,), jnp.float32),
            pltpu.VMEM((LANES, 2 * EMBED_DIM), jnp.float32),
            pltpu.VMEM((LANES, 2 * EMBED_DIM), jnp.float32),
            pltpu.VMEM((LANES, 2 * EMBED_DIM), jnp.float32),
            pltpu.VMEM((LANES, 2 * EMBED_DIM), jnp.float32),
            pltpu.SemaphoreType.DMA,
            pltpu.SemaphoreType.DMA,
            pltpu.SemaphoreType.DMA,
            pltpu.SemaphoreType.DMA,
        ],
    )(cidx, pidx, ctab2, ptab2)
    return padded.reshape(NUM_WORKERS, OUT_STRIDE)[:, :B_PER_W].reshape(-1)


def kernel(customer, product, customer_table, product_table):
    return _run(customer, product, customer_table, product_table)


# speedup vs baseline: 10.6617x; 1.4440x over previous
import jax
import jax.numpy as jnp
from jax import lax
from jax.experimental import pallas as pl
from jax.experimental.pallas import tpu as pltpu
from jax.experimental.pallas import tpu_sc as plsc

NUM_CORES = 2
NUM_SUBCORES = 16
LANES = 16
NUM_WORKERS = NUM_CORES * NUM_SUBCORES

NUM_CUSTOMERS = 1000000
NUM_PRODUCTS = 100000
BATCH = 16384
EMBED_DIM = 64
B_PER_W = BATCH // NUM_WORKERS
GROUPS = B_PER_W // LANES
OUT_STRIDE = 2 * B_PER_W


def _idx_vec(idx_s, g):
    e = g * LANES + lax.iota(jnp.int32, LANES)
    return plsc.load_gather(idx_s, [lax.shift_right_logical(e, 7),
                                    jnp.bitwise_and(e, 127)])


def _fetch_grp(tab, idx_s, g, buf, sem):
    grp = lax.shift_right_logical(_idx_vec(idx_s, g), 3)
    for l in range(LANES):
        base8 = pl.multiple_of(grp[l] * 8, 8)
        pltpu.async_copy(tab.at[pl.ds(base8, 8), :], buf.at[l], sem)


def _wait_grp(tab, idx_s, g, buf, sem):
    grp = lax.shift_right_logical(_idx_vec(idx_s, g), 3)
    for l in range(LANES):
        base8 = pl.multiple_of(grp[l] * 8, 8)
        pltpu.make_async_copy(tab.at[pl.ds(base8, 8), :],
                              buf.at[l], sem).wait()


def _dot(cidx_s, pidx_s, g, cbuf, pbuf, out_v):
    lane = lax.iota(jnp.int32, LANES)
    csub = jnp.bitwise_and(_idx_vec(cidx_s, g), 7)
    psub = jnp.bitwise_and(_idx_vec(pidx_s, g), 7)
    acc = jnp.zeros((LANES,), jnp.float32)
    for d in range(EMBED_DIM):
        dv = jnp.full((LANES,), d, jnp.int32)
        cv = plsc.load_gather(cbuf, [lane, csub, dv])
        pv = plsc.load_gather(pbuf, [lane, psub, dv])
        acc = acc + cv * pv
    plsc.store_scatter(out_v, [g * LANES + lane], acc)


def _sc_body(cidx_hbm, pidx_hbm, ctab, ptab, out_hbm,
             cidx_s, pidx_s, out_v,
             cbuf_a, pbuf_a, cbuf_b, pbuf_b,
             csem_a, psem_a, csem_b, psem_b):
    wid = lax.axis_index("s") * NUM_CORES + lax.axis_index("c")

    pltpu.sync_copy(cidx_hbm.at[wid], cidx_s)
    pltpu.sync_copy(pidx_hbm.at[wid], pidx_s)

    def pair_body(t, carry):
        g0 = 2 * t
        g1 = 2 * t + 1
        _fetch_grp(ctab, cidx_s, g0, cbuf_a, csem_a)
        _fetch_grp(ptab, pidx_s, g0, pbuf_a, psem_a)
        _fetch_grp(ctab, cidx_s, g1, cbuf_b, csem_b)
        _fetch_grp(ptab, pidx_s, g1, pbuf_b, psem_b)
        _wait_grp(ctab, cidx_s, g0, cbuf_a, csem_a)
        _wait_grp(ptab, pidx_s, g0, pbuf_a, psem_a)
        _dot(cidx_s, pidx_s, g0, cbuf_a, pbuf_a, out_v)
        _wait_grp(ctab, cidx_s, g1, cbuf_b, csem_b)
        _wait_grp(ptab, pidx_s, g1, pbuf_b, psem_b)
        _dot(cidx_s, pidx_s, g1, cbuf_b, pbuf_b, out_v)
        return carry

    lax.fori_loop(0, GROUPS // 2, pair_body, 0)

    pltpu.sync_copy(out_v, out_hbm.at[pl.ds(wid * OUT_STRIDE, OUT_STRIDE)])


@jax.jit
def _run(customer, product, customer_table, product_table):
    mesh = plsc.VectorSubcoreMesh(core_axis_name="c", subcore_axis_name="s",
                                  num_cores=NUM_CORES,
                                  num_subcores=NUM_SUBCORES)
    cidx = jnp.pad(customer.reshape(NUM_WORKERS, 4, 128),
                   ((0, 0), (0, 4), (0, 0)))
    pidx = jnp.pad(product.reshape(NUM_WORKERS, 4, 128),
                   ((0, 0), (0, 4), (0, 0)))
    padded = pl.kernel(
        _sc_body,
        out_type=jax.ShapeDtypeStruct((NUM_WORKERS * OUT_STRIDE,),
                                      jnp.float32),
        mesh=mesh,
        compiler_params=pltpu.CompilerParams(needs_layout_passes=False),
        scratch_types=[
            pltpu.VMEM((8, 128), jnp.int32),
            pltpu.VMEM((8, 128), jnp.int32),
            pltpu.VMEM((OUT_STRIDE,), jnp.float32),
            pltpu.VMEM((LANES, 8, EMBED_DIM), jnp.float32),
            pltpu.VMEM((LANES, 8, EMBED_DIM), jnp.float32),
            pltpu.VMEM((LANES, 8, EMBED_DIM), jnp.float32),
            pltpu.VMEM((LANES, 8, EMBED_DIM), jnp.float32),
            pltpu.SemaphoreType.DMA,
            pltpu.SemaphoreType.DMA,
            pltpu.SemaphoreType.DMA,
            pltpu.SemaphoreType.DMA,
        ],
    )(cidx, pidx, customer_table, product_table)
    return padded.reshape(NUM_WORKERS, OUT_STRIDE)[:, :B_PER_W].reshape(-1)


def kernel(customer, product, customer_table, product_table):
    return _run(customer, product, customer_table, product_table)


# speedup vs baseline: 16.3518x; 1.5337x over previous
import jax
import jax.numpy as jnp
from jax import lax
from jax.experimental import pallas as pl
from jax.experimental.pallas import tpu as pltpu
from jax.experimental.pallas import tpu_sc as plsc

NUM_CORES = 2
NUM_SUBCORES = 16
LANES = 16
NUM_WORKERS = NUM_CORES * NUM_SUBCORES

NUM_CUSTOMERS = 1000000
NUM_PRODUCTS = 100000
BATCH = 16384
EMBED_DIM = 64
B_PER_W = BATCH // NUM_WORKERS
GROUPS = B_PER_W // LANES
OUT_STRIDE = 2 * B_PER_W
SLOT = 1024

NBLK = NUM_CUSTOMERS // 128
TAIL0 = NBLK * 128
BLK_PER_W = (NBLK + NUM_WORKERS - 1) // NUM_WORKERS
BUCKET_CAP = 8
SC_CELL = BLK_PER_W + 1


def _evec(i):
    e = i * LANES + lax.iota(jnp.int32, LANES)
    return e, [lax.shift_right_logical(e, 9),
               jnp.bitwise_and(lax.shift_right_logical(e, 7), 3),
               jnp.bitwise_and(e, 127)]


def _scalar(ref, i):
    v = plsc.load_gather(ref, [jnp.full((LANES,), i, jnp.int32)])
    return v[0]


def _extract_col(chunk, j, stage, slot):
    jv = jnp.full((LANES,), j, jnp.int32)
    lane = lax.iota(jnp.int32, LANES)
    for q in range(4):
        rows = q * LANES + lane
        v = plsc.load_gather(chunk, [rows, jv])
        plsc.store_scatter(stage, [slot * 128 + q * LANES + lane], v)


def _gather_body(cidx_hbm, ctab_t, cgat_hbm,
                 cidx_v, ent_ex, ent_cust, buckets_ex, buckets_j,
                 ovf_ex, ovf_cust, chunk_a, chunk_b, stage,
                 counts_s, csem_a, csem_b, osem):
    wid = lax.axis_index("s") * NUM_CORES + lax.axis_index("c")
    blk0 = wid * BLK_PER_W
    nblk_w = jnp.minimum(jnp.int32(NBLK) - blk0, BLK_PER_W)
    nblk_w = jnp.maximum(nblk_w, 0)

    pltpu.sync_copy(cidx_hbm, cidx_v)

    def scan_step(i, cnt):
        e, dims = _evec(i)
        cust = plsc.load_gather(cidx_v, dims)
        blk = lax.shift_right_logical(cust, 7)
        mask = (blk >= blk0) & (blk < blk0 + nblk_w)
        pos = cnt + plsc.cumsum(mask.astype(jnp.int32)) - 1
        plsc.store_scatter(ent_ex, [pos], e, mask=mask)
        plsc.store_scatter(ent_cust, [pos], cust, mask=mask)
        return cnt + plsc.all_reduce_population_count(mask)[0]

    n_ent = lax.fori_loop(0, BATCH // LANES, scan_step, jnp.int32(0))

    def zero_step(i, c):
        counts_s[i] = 0
        return c
    lax.fori_loop(0, BLK_PER_W + 2, zero_step, 0)

    def bucket_step(i, c):
        cust = _scalar(ent_cust, i)
        ex = _scalar(ent_ex, i)
        loc = lax.shift_right_logical(cust, 7) - blk0
        j = jnp.bitwise_and(cust, 127)
        p = counts_s[loc]

        @pl.when(p < BUCKET_CAP)
        def _():
            slot = loc * BUCKET_CAP + p
            plsc.store_scatter(buckets_ex, [jnp.full((LANES,), slot,
                                                     jnp.int32)],
                               jnp.full((LANES,), ex, jnp.int32),
                               mask=lax.iota(jnp.int32, LANES) == 0)
            plsc.store_scatter(buckets_j, [jnp.full((LANES,), slot,
                                                    jnp.int32)],
                               jnp.full((LANES,), j, jnp.int32),
                               mask=lax.iota(jnp.int32, LANES) == 0)
            counts_s[loc] = p + 1

        @pl.when(p >= BUCKET_CAP)
        def _():
            q = counts_s[BLK_PER_W]
            plsc.store_scatter(ovf_ex, [jnp.full((LANES,), q, jnp.int32)],
                               jnp.full((LANES,), ex, jnp.int32),
                               mask=lax.iota(jnp.int32, LANES) == 0)
            plsc.store_scatter(ovf_cust, [jnp.full((LANES,), q, jnp.int32)],
                               jnp.full((LANES,), cust, jnp.int32),
                               mask=lax.iota(jnp.int32, LANES) == 0)
            counts_s[BLK_PER_W] = q + 1
        return c

    lax.fori_loop(0, n_ent, bucket_step, 0)

    def start_fetch(b, buf, sem):
        off = pl.multiple_of((blk0 + b) * 128, 128)
        pltpu.async_copy(ctab_t.at[:, pl.ds(off, 128)], buf, sem)

    def wait_fetch(b, buf, sem):
        pltpu.make_async_copy(ctab_t.at[:, pl.ds(0, 128)], buf, sem).wait()

    def process_block(b, buf):
        nloc = jnp.minimum(counts_s[b], BUCKET_CAP)
        blk = blk0 + b

        def one(k, c):
            ex = _scalar(buckets_ex, b * BUCKET_CAP + k)
            j = _scalar(buckets_j, b * BUCKET_CAP + k)
            sc = counts_s[SC_CELL]
            ss = jnp.bitwise_and(sc, 7)

            @pl.when(sc >= 8)
            def _():
                pltpu.make_async_copy(stage.at[pl.ds(0, 128)],
                                      cgat_hbm.at[pl.ds(0, 128)],
                                      osem).wait()
            _extract_col(buf, j, stage, ss)
            pltpu.async_copy(
                stage.at[pl.ds(ss * 128, 128)],
                cgat_hbm.at[pl.ds(pl.multiple_of(ex * SLOT, 128), 128)],
                osem)
            counts_s[SC_CELL] = sc + 1
            return c

        lax.fori_loop(0, nloc, one, 0)

    start_fetch(0, chunk_a, csem_a)

    def sweep2(t, carry):
        b0 = 2 * t
        b1 = 2 * t + 1

        @pl.when(b1 < nblk_w)
        def _():
            start_fetch(b1, chunk_b, csem_b)

        @pl.when(b0 < nblk_w)
        def _():
            wait_fetch(b0, chunk_a, csem_a)
            process_block(b0, chunk_a)

            @pl.when(b0 + 2 < nblk_w)
            def _():
                start_fetch(b0 + 2, chunk_a, csem_a)

        @pl.when(b1 < nblk_w)
        def _():
            wait_fetch(b1, chunk_b, csem_b)
            process_block(b1, chunk_b)
        return carry

    lax.fori_loop(0, (BLK_PER_W + 1) // 2, sweep2, 0)

    def ovf_step(i, c):
        cust = _scalar(ovf_cust, i)
        ex = _scalar(ovf_ex, i)
        blk = lax.shift_right_logical(cust, 7)
        j = jnp.bitwise_and(cust, 127)
        sc = counts_s[SC_CELL]
        ss = jnp.bitwise_and(sc, 7)

        @pl.when(sc >= 8)
        def _():
            pltpu.make_async_copy(stage.at[pl.ds(0, 128)],
                                  cgat_hbm.at[pl.ds(0, 128)], osem).wait()

        off = pl.multiple_of(blk * 128, 128)
        pltpu.sync_copy(ctab_t.at[:, pl.ds(off, 128)], chunk_a)
        _extract_col(chunk_a, j, stage, ss)
        pltpu.async_copy(
            stage.at[pl.ds(ss * 128, 128)],
            cgat_hbm.at[pl.ds(pl.multiple_of(ex * SLOT, 128), 128)], osem)
        counts_s[SC_CELL] = sc + 1
        return c

    lax.fori_loop(0, counts_s[BLK_PER_W], ovf_step, 0)

    n_out = jnp.minimum(counts_s[SC_CELL], 8)

    def drain_step(i, c):
        @pl.when(i < n_out)
        def _():
            pltpu.make_async_copy(stage.at[pl.ds(0, 128)],
                                  cgat_hbm.at[pl.ds(0, 128)], osem).wait()
        return c

    lax.fori_loop(0, 8, drain_step, 0)


def _idx_vec(idx_s, g):
    e = g * LANES + lax.iota(jnp.int32, LANES)
    return plsc.load_gather(idx_s, [lax.shift_right_logical(e, 7),
                                    jnp.bitwise_and(e, 127)])


def _fetch_grp(tab, idx_s, g, buf, sem):
    grp = lax.shift_right_logical(_idx_vec(idx_s, g), 3)
    for l in range(LANES):
        base8 = pl.multiple_of(grp[l] * 8, 8)
        pltpu.async_copy(tab.at[pl.ds(base8, 8), :], buf.at[l], sem)


def _wait_grp(tab, idx_s, g, buf, sem):
    grp = lax.shift_right_logical(_idx_vec(idx_s, g), 3)
    for l in range(LANES):
        base8 = pl.multiple_of(grp[l] * 8, 8)
        pltpu.make_async_copy(tab.at[pl.ds(base8, 8), :],
                              buf.at[l], sem).wait()


def _fetch_cust(cgat_hbm, wid, g, cbuf, sem):
    base = (wid * B_PER_W + g * LANES) * SLOT
    pltpu.async_copy(cgat_hbm.at[pl.ds(pl.multiple_of(base, SLOT),
                                       LANES * SLOT)], cbuf, sem)


def _dot_body(cidx_hbm, pidx_hbm, ptab, ctail_hbm, cgat_hbm, out_hbm,
              cidx_s, pidx_s, out_v, ctail_v, cbuf_a, cbuf_b,
              pbuf_a, pbuf_b, csem_a, csem_b, psem_a, psem_b):
    wid = lax.axis_index("s") * NUM_CORES + lax.axis_index("c")

    pltpu.sync_copy(cidx_hbm.at[wid], cidx_s)
    pltpu.sync_copy(pidx_hbm.at[wid], pidx_s)
    pltpu.sync_copy(ctail_hbm, ctail_v)

    lane = lax.iota(jnp.int32, LANES)

    def dot(g, cbuf, pbuf):
        cust = _idx_vec(cidx_s, g)
        tail = cust >= TAIL0
        trow = jnp.maximum(cust - TAIL0, 0)
        psub = jnp.bitwise_and(_idx_vec(pidx_s, g), 7)
        acc = jnp.zeros((LANES,), jnp.float32)
        for d in range(EMBED_DIM):
            dv = jnp.full((LANES,), d, jnp.int32)
            cv = plsc.load_gather(cbuf, [lane * SLOT + dv])
            tv = plsc.load_gather(ctail_v, [trow, dv])
            cv = jnp.where(tail, tv, cv)
            pv = plsc.load_gather(pbuf, [lane, psub, dv])
            acc = acc + cv * pv
        plsc.store_scatter(out_v, [g * LANES + lane], acc)

    def pair_body(t, carry):
        g0 = 2 * t
        g1 = 2 * t + 1
        _fetch_cust(cgat_hbm, wid, g0, cbuf_a, csem_a)
        _fetch_grp(ptab, pidx_s, g0, pbuf_a, psem_a)
        _fetch_cust(cgat_hbm, wid, g1, cbuf_b, csem_b)
        _fetch_grp(ptab, pidx_s, g1, pbuf_b, psem_b)
        pltpu.make_async_copy(cgat_hbm.at[pl.ds(0, LANES * SLOT)], cbuf_a,
                              csem_a).wait()
        _wait_grp(ptab, pidx_s, g0, pbuf_a, psem_a)
        dot(g0, cbuf_a, pbuf_a)
        pltpu.make_async_copy(cgat_hbm.at[pl.ds(0, LANES * SLOT)], cbuf_b,
                              csem_b).wait()
        _wait_grp(ptab, pidx_s, g1, pbuf_b, psem_b)
        dot(g1, cbuf_b, pbuf_b)
        return carry

    lax.fori_loop(0, GROUPS // 2, pair_body, 0)

    pltpu.sync_copy(out_v, out_hbm.at[pl.ds(wid * OUT_STRIDE, OUT_STRIDE)])


@jax.jit
def _run(customer, product, customer_table, product_table):
    mesh = plsc.VectorSubcoreMesh(core_axis_name="c", subcore_axis_name="s",
                                  num_cores=NUM_CORES,
                                  num_subcores=NUM_SUBCORES)
    cidx = jnp.pad(customer.reshape(NUM_WORKERS, 4, 128),
                   ((0, 0), (0, 4), (0, 0)))
    pidx = jnp.pad(product.reshape(NUM_WORKERS, 4, 128),
                   ((0, 0), (0, 4), (0, 0)))

    cgat = pl.kernel(
        _gather_body,
        out_type=jax.ShapeDtypeStruct((BATCH * SLOT,), jnp.float32),
        mesh=mesh,
        compiler_params=pltpu.CompilerParams(needs_layout_passes=False),
        scratch_types=[
            pltpu.VMEM((NUM_WORKERS, 8, 128), jnp.int32),
            pltpu.VMEM((BATCH,), jnp.int32),
            pltpu.VMEM((BATCH,), jnp.int32),
            pltpu.VMEM(((BLK_PER_W + 1) * BUCKET_CAP,), jnp.int32),
            pltpu.VMEM(((BLK_PER_W + 1) * BUCKET_CAP,), jnp.int32),
            pltpu.VMEM((BATCH,), jnp.int32),
            pltpu.VMEM((BATCH,), jnp.int32),
            pltpu.VMEM((EMBED_DIM, 128), jnp.float32),
            pltpu.VMEM((EMBED_DIM, 128), jnp.float32),
            pltpu.VMEM((8 * 128,), jnp.float32),
            pltpu.SMEM((BLK_PER_W + 2,), jnp.int32),
            pltpu.SemaphoreType.DMA,
            pltpu.SemaphoreType.DMA,
            pltpu.SemaphoreType.DMA,
        ],
    )(cidx, customer_table.T)

    padded = pl.kernel(
        _dot_body,
        out_type=jax.ShapeDtypeStruct((NUM_WORKERS * OUT_STRIDE,),
                                      jnp.float32),
        mesh=mesh,
        compiler_params=pltpu.CompilerParams(needs_layout_passes=False),
        scratch_types=[
            pltpu.VMEM((8, 128), jnp.int32),
            pltpu.VMEM((8, 128), jnp.int32),
            pltpu.VMEM((OUT_STRIDE,), jnp.float32),
            pltpu.VMEM((EMBED_DIM, EMBED_DIM), jnp.float32),
            pltpu.VMEM((LANES * SLOT,), jnp.float32),
            pltpu.VMEM((LANES * SLOT,), jnp.float32),
            pltpu.VMEM((LANES, 8, EMBED_DIM), jnp.float32),
            pltpu.VMEM((LANES, 8, EMBED_DIM), jnp.float32),
            pltpu.SemaphoreType.DMA,
            pltpu.SemaphoreType.DMA,
            pltpu.SemaphoreType.DMA,
            pltpu.SemaphoreType.DMA,
        ],
    )(cidx, pidx, product_table, customer_table[TAIL0:], cgat)
    return padded.reshape(NUM_WORKERS, OUT_STRIDE)[:, :B_PER_W].reshape(-1)


def kernel(customer, product, customer_table, product_table):
    return _run(customer, product, customer_table, product_table)


# speedup vs baseline: 17.3610x; 1.0617x over previous
import jax
import jax.numpy as jnp
from jax import lax
from jax.experimental import pallas as pl
from jax.experimental.pallas import tpu as pltpu
from jax.experimental.pallas import tpu_sc as plsc

NUM_CORES = 2
NUM_SUBCORES = 16
LANES = 16
NUM_WORKERS = NUM_CORES * NUM_SUBCORES

NUM_CUSTOMERS = 1000000
NUM_PRODUCTS = 100000
BATCH = 16384
EMBED_DIM = 64
B_PER_W = BATCH // NUM_WORKERS
GROUPS = B_PER_W // LANES
OUT_STRIDE = 2 * B_PER_W
SLOT = 128

NBLK = NUM_CUSTOMERS // 128
TAIL0 = NBLK * 128
BLK_PER_W = (NBLK + NUM_WORKERS - 1) // NUM_WORKERS
BUCKET_CAP = 8
SC_CELL = BLK_PER_W + 1


def _evec(i):
    e = i * LANES + lax.iota(jnp.int32, LANES)
    return e, [lax.shift_right_logical(e, 9),
               jnp.bitwise_and(lax.shift_right_logical(e, 7), 3),
               jnp.bitwise_and(e, 127)]


def _scalar(ref, i):
    v = plsc.load_gather(ref, [jnp.full((LANES,), i, jnp.int32)])
    return v[0]


def _extract_col(chunk, j, stage, slot):
    jv = jnp.full((LANES,), j, jnp.int32)
    lane = lax.iota(jnp.int32, LANES)
    for q in range(4):
        rows = q * LANES + lane
        v = plsc.load_gather(chunk, [rows, jv])
        plsc.store_scatter(stage, [slot * 128 + q * LANES + lane], v)


def _gather_body(cidx_hbm, ctab_t, cgat_hbm,
                 cidx_v, ent_ex, ent_cust, buckets_ex, buckets_j,
                 ovf_ex, ovf_cust, chunk_a, chunk_b, stage,
                 counts_s, csem_a, csem_b, osem):
    wid = lax.axis_index("s") * NUM_CORES + lax.axis_index("c")
    blk0 = wid * BLK_PER_W
    nblk_w = jnp.minimum(jnp.int32(NBLK) - blk0, BLK_PER_W)
    nblk_w = jnp.maximum(nblk_w, 0)

    pltpu.sync_copy(cidx_hbm, cidx_v)

    def scan_step(i, cnt):
        e, dims = _evec(i)
        cust = plsc.load_gather(cidx_v, dims)
        blk = lax.shift_right_logical(cust, 7)
        mask = (blk >= blk0) & (blk < blk0 + nblk_w)
        pos = cnt + plsc.cumsum(mask.astype(jnp.int32)) - 1
        plsc.store_scatter(ent_ex, [pos], e, mask=mask)
        plsc.store_scatter(ent_cust, [pos], cust, mask=mask)
        return cnt + plsc.all_reduce_population_count(mask)[0]

    n_ent = lax.fori_loop(0, BATCH // LANES, scan_step, jnp.int32(0))

    def zero_step(i, c):
        counts_s[i] = 0
        return c
    lax.fori_loop(0, BLK_PER_W + 2, zero_step, 0)

    def bucket_step(i, c):
        cust = _scalar(ent_cust, i)
        ex = _scalar(ent_ex, i)
        loc = lax.shift_right_logical(cust, 7) - blk0
        j = jnp.bitwise_and(cust, 127)
        p = counts_s[loc]

        @pl.when(p < BUCKET_CAP)
        def _():
            slot = loc * BUCKET_CAP + p
            plsc.store_scatter(buckets_ex, [jnp.full((LANES,), slot,
                                                     jnp.int32)],
                               jnp.full((LANES,), ex, jnp.int32),
                               mask=lax.iota(jnp.int32, LANES) == 0)
            plsc.store_scatter(buckets_j, [jnp.full((LANES,), slot,
                                                    jnp.int32)],
                               jnp.full((LANES,), j, jnp.int32),
                               mask=lax.iota(jnp.int32, LANES) == 0)
            counts_s[loc] = p + 1

        @pl.when(p >= BUCKET_CAP)
        def _():
            q = counts_s[BLK_PER_W]
            plsc.store_scatter(ovf_ex, [jnp.full((LANES,), q, jnp.int32)],
                               jnp.full((LANES,), ex, jnp.int32),
                               mask=lax.iota(jnp.int32, LANES) == 0)
            plsc.store_scatter(ovf_cust, [jnp.full((LANES,), q, jnp.int32)],
                               jnp.full((LANES,), cust, jnp.int32),
                               mask=lax.iota(jnp.int32, LANES) == 0)
            counts_s[BLK_PER_W] = q + 1
        return c

    lax.fori_loop(0, n_ent, bucket_step, 0)

    def start_fetch(b, buf, sem):
        off = pl.multiple_of((blk0 + b) * 128, 128)
        pltpu.async_copy(ctab_t.at[:, pl.ds(off, 128)], buf, sem)

    def wait_fetch(b, buf, sem):
        pltpu.make_async_copy(ctab_t.at[:, pl.ds(0, 128)], buf, sem).wait()

    def process_block(b, buf):
        nloc = jnp.minimum(counts_s[b], BUCKET_CAP)
        blk = blk0 + b

        def one(k, c):
            ex = _scalar(buckets_ex, b * BUCKET_CAP + k)
            j = _scalar(buckets_j, b * BUCKET_CAP + k)
            sc = counts_s[SC_CELL]
            ss = jnp.bitwise_and(sc, 7)

            @pl.when(sc >= 8)
            def _():
                pltpu.make_async_copy(stage.at[pl.ds(0, 128)],
                                      cgat_hbm.at[pl.ds(0, 128)],
                                      osem).wait()
            _extract_col(buf, j, stage, ss)
            pltpu.async_copy(
                stage.at[pl.ds(ss * 128, 128)],
                cgat_hbm.at[pl.ds(pl.multiple_of(ex * SLOT, 128), 128)],
                osem)
            counts_s[SC_CELL] = sc + 1
            return c

        lax.fori_loop(0, nloc, one, 0)

    start_fetch(0, chunk_a, csem_a)

    def sweep2(t, carry):
        b0 = 2 * t
        b1 = 2 * t + 1

        @pl.when(b1 < nblk_w)
        def _():
            start_fetch(b1, chunk_b, csem_b)

        @pl.when(b0 < nblk_w)
        def _():
            wait_fetch(b0, chunk_a, csem_a)
            process_block(b0, chunk_a)

            @pl.when(b0 + 2 < nblk_w)
            def _():
                start_fetch(b0 + 2, chunk_a, csem_a)

        @pl.when(b1 < nblk_w)
        def _():
            wait_fetch(b1, chunk_b, csem_b)
            process_block(b1, chunk_b)
        return carry

    lax.fori_loop(0, (BLK_PER_W + 1) // 2, sweep2, 0)

    def ovf_step(i, c):
        cust = _scalar(ovf_cust, i)
        ex = _scalar(ovf_ex, i)
        blk = lax.shift_right_logical(cust, 7)
        j = jnp.bitwise_and(cust, 127)
        sc = counts_s[SC_CELL]
        ss = jnp.bitwise_and(sc, 7)

        @pl.when(sc >= 8)
        def _():
            pltpu.make_async_copy(stage.at[pl.ds(0, 128)],
                                  cgat_hbm.at[pl.ds(0, 128)], osem).wait()

        off = pl.multiple_of(blk * 128, 128)
        pltpu.sync_copy(ctab_t.at[:, pl.ds(off, 128)], chunk_a)
        _extract_col(chunk_a, j, stage, ss)
        pltpu.async_copy(
            stage.at[pl.ds(ss * 128, 128)],
            cgat_hbm.at[pl.ds(pl.multiple_of(ex * SLOT, 128), 128)], osem)
        counts_s[SC_CELL] = sc + 1
        return c

    lax.fori_loop(0, counts_s[BLK_PER_W], ovf_step, 0)

    n_out = jnp.minimum(counts_s[SC_CELL], 8)

    def drain_step(i, c):
        @pl.when(i < n_out)
        def _():
            pltpu.make_async_copy(stage.at[pl.ds(0, 128)],
                                  cgat_hbm.at[pl.ds(0, 128)], osem).wait()
        return c

    lax.fori_loop(0, 8, drain_step, 0)


def _idx_vec(idx_s, g):
    e = g * LANES + lax.iota(jnp.int32, LANES)
    return plsc.load_gather(idx_s, [lax.shift_right_logical(e, 7),
                                    jnp.bitwise_and(e, 127)])


def _fetch_grp(tab, idx_s, g, buf, sem):
    grp = lax.shift_right_logical(_idx_vec(idx_s, g), 3)
    for l in range(LANES):
        base8 = pl.multiple_of(grp[l] * 8, 8)
        pltpu.async_copy(tab.at[pl.ds(base8, 8), :], buf.at[l], sem)


def _wait_grp(tab, idx_s, g, buf, sem):
    grp = lax.shift_right_logical(_idx_vec(idx_s, g), 3)
    for l in range(LANES):
        base8 = pl.multiple_of(grp[l] * 8, 8)
        pltpu.make_async_copy(tab.at[pl.ds(base8, 8), :],
                              buf.at[l], sem).wait()


def _fetch_cust(cgat_hbm, wid, g, cbuf, sem):
    base = (wid * B_PER_W + g * LANES) * SLOT
    pltpu.async_copy(cgat_hbm.at[pl.ds(pl.multiple_of(base, SLOT),
                                       LANES * SLOT)], cbuf, sem)


def _dot_body(cidx_hbm, pidx_hbm, ptab, ctail_hbm, cgat_hbm, out_hbm,
              cidx_s, pidx_s, out_v, ctail_v, cbuf_a, cbuf_b,
              pbuf_a, pbuf_b, csem_a, csem_b, psem_a, psem_b):
    wid = lax.axis_index("s") * NUM_CORES + lax.axis_index("c")

    pltpu.sync_copy(cidx_hbm.at[wid], cidx_s)
    pltpu.sync_copy(pidx_hbm.at[wid], pidx_s)
    pltpu.sync_copy(ctail_hbm, ctail_v)

    lane = lax.iota(jnp.int32, LANES)

    def dot(g, cbuf, pbuf):
        cust = _idx_vec(cidx_s, g)
        tail = cust >= TAIL0
        trow = jnp.maximum(cust - TAIL0, 0)
        psub = jnp.bitwise_and(_idx_vec(pidx_s, g), 7)
        acc = jnp.zeros((LANES,), jnp.float32)
        for d in range(EMBED_DIM):
            dv = jnp.full((LANES,), d, jnp.int32)
            cv = plsc.load_gather(cbuf, [lane * SLOT + dv])
            tv = plsc.load_gather(ctail_v, [trow, dv])
            cv = jnp.where(tail, tv, cv)
            pv = plsc.load_gather(pbuf, [lane, psub, dv])
            acc = acc + cv * pv
        plsc.store_scatter(out_v, [g * LANES + lane], acc)

    def pair_body(t, carry):
        g0 = 2 * t
        g1 = 2 * t + 1
        _fetch_cust(cgat_hbm, wid, g0, cbuf_a, csem_a)
        _fetch_grp(ptab, pidx_s, g0, pbuf_a, psem_a)
        _fetch_cust(cgat_hbm, wid, g1, cbuf_b, csem_b)
        _fetch_grp(ptab, pidx_s, g1, pbuf_b, psem_b)
        pltpu.make_async_copy(cgat_hbm.at[pl.ds(0, LANES * SLOT)], cbuf_a,
                              csem_a).wait()
        _wait_grp(ptab, pidx_s, g0, pbuf_a, psem_a)
        dot(g0, cbuf_a, pbuf_a)
        pltpu.make_async_copy(cgat_hbm.at[pl.ds(0, LANES * SLOT)], cbuf_b,
                              csem_b).wait()
        _wait_grp(ptab, pidx_s, g1, pbuf_b, psem_b)
        dot(g1, cbuf_b, pbuf_b)
        return carry

    lax.fori_loop(0, GROUPS // 2, pair_body, 0)

    pltpu.sync_copy(out_v, out_hbm.at[pl.ds(wid * OUT_STRIDE, OUT_STRIDE)])


@jax.jit
def _run(customer, product, customer_table, product_table):
    mesh = plsc.VectorSubcoreMesh(core_axis_name="c", subcore_axis_name="s",
                                  num_cores=NUM_CORES,
                                  num_subcores=NUM_SUBCORES)
    cidx = jnp.pad(customer.reshape(NUM_WORKERS, 4, 128),
                   ((0, 0), (0, 4), (0, 0)))
    pidx = jnp.pad(product.reshape(NUM_WORKERS, 4, 128),
                   ((0, 0), (0, 4), (0, 0)))

    cgat = pl.kernel(
        _gather_body,
        out_type=jax.ShapeDtypeStruct((BATCH * SLOT,), jnp.float32),
        mesh=mesh,
        compiler_params=pltpu.CompilerParams(needs_layout_passes=False),
        scratch_types=[
            pltpu.VMEM((NUM_WORKERS, 8, 128), jnp.int32),
            pltpu.VMEM((BATCH,), jnp.int32),
            pltpu.VMEM((BATCH,), jnp.int32),
            pltpu.VMEM(((BLK_PER_W + 1) * BUCKET_CAP,), jnp.int32),
            pltpu.VMEM(((BLK_PER_W + 1) * BUCKET_CAP,), jnp.int32),
            pltpu.VMEM((BATCH,), jnp.int32),
            pltpu.VMEM((BATCH,), jnp.int32),
            pltpu.VMEM((EMBED_DIM, 128), jnp.float32),
            pltpu.VMEM((EMBED_DIM, 128), jnp.float32),
            pltpu.VMEM((8 * 128,), jnp.float32),
            pltpu.SMEM((BLK_PER_W + 2,), jnp.int32),
            pltpu.SemaphoreType.DMA,
            pltpu.SemaphoreType.DMA,
            pltpu.SemaphoreType.DMA,
        ],
    )(cidx, customer_table.T)

    padded = pl.kernel(
        _dot_body,
        out_type=jax.ShapeDtypeStruct((NUM_WORKERS * OUT_STRIDE,),
                                      jnp.float32),
        mesh=mesh,
        compiler_params=pltpu.CompilerParams(needs_layout_passes=False),
        scratch_types=[
            pltpu.VMEM((8, 128), jnp.int32),
            pltpu.VMEM((8, 128), jnp.int32),
            pltpu.VMEM((OUT_STRIDE,), jnp.float32),
            pltpu.VMEM((EMBED_DIM, EMBED_DIM), jnp.float32),
            pltpu.VMEM((LANES * SLOT,), jnp.float32),
            pltpu.VMEM((LANES * SLOT,), jnp.float32),
            pltpu.VMEM((LANES, 8, EMBED_DIM), jnp.float32),
            pltpu.VMEM((LANES, 8, EMBED_DIM), jnp.float32),
            pltpu.SemaphoreType.DMA,
            pltpu.SemaphoreType.DMA,
            pltpu.SemaphoreType.DMA,
            pltpu.SemaphoreType.DMA,
        ],
    )(cidx, pidx, product_table, customer_table[TAIL0:], cgat)
    return padded.reshape(NUM_WORKERS, OUT_STRIDE)[:, :B_PER_W].reshape(-1)


def kernel(customer, product, customer_table, product_table):
    return _run(customer, product, customer_table, product_table)


# speedup vs baseline: 17.5465x; 1.0107x over previous
import jax
import jax.numpy as jnp
from jax import lax
from jax.experimental import pallas as pl
from jax.experimental.pallas import tpu as pltpu
from jax.experimental.pallas import tpu_sc as plsc

NUM_CORES = 2
NUM_SUBCORES = 16
LANES = 16
NUM_WORKERS = NUM_CORES * NUM_SUBCORES

NUM_CUSTOMERS = 1000000
NUM_PRODUCTS = 100000
BATCH = 16384
EMBED_DIM = 64
B_PER_W = BATCH // NUM_WORKERS
GROUPS = B_PER_W // LANES
OUT_STRIDE = 2 * B_PER_W
SLOT = 128

NBLK = NUM_CUSTOMERS // 128
TAIL0 = NBLK * 128
BLK_PER_W = (NBLK + NUM_WORKERS - 1) // NUM_WORKERS
BUCKET_CAP = 8
SC_CELL = BLK_PER_W + 1


def _evec(i):
    e = i * LANES + lax.iota(jnp.int32, LANES)
    return e, [lax.shift_right_logical(e, 9),
               jnp.bitwise_and(lax.shift_right_logical(e, 7), 3),
               jnp.bitwise_and(e, 127)]


def _scalar(ref, i):
    v = plsc.load_gather(ref, [jnp.full((LANES,), i, jnp.int32)])
    return v[0]


def _extract_col(chunk, j, stage, slot):
    jv = jnp.full((LANES,), j, jnp.int32)
    lane = lax.iota(jnp.int32, LANES)
    for q in range(4):
        rows = q * LANES + lane
        v = plsc.load_gather(chunk, [rows, jv])
        plsc.store_scatter(stage, [slot * 128 + q * LANES + lane], v)


def _gather_body(cidx_hbm, ctab_t, cgat_hbm,
                 cidx_v, ent_ex, ent_cust, buckets_ex, buckets_j,
                 ovf_ex, ovf_cust, chunk_a, chunk_b, stage,
                 counts_s, csem_a, csem_b, osem):
    wid = lax.axis_index("s") * NUM_CORES + lax.axis_index("c")
    blk0 = wid * BLK_PER_W
    nblk_w = jnp.minimum(jnp.int32(NBLK) - blk0, BLK_PER_W)
    nblk_w = jnp.maximum(nblk_w, 0)

    pltpu.sync_copy(cidx_hbm, cidx_v)

    def scan_step(i, cnt):
        e, dims = _evec(i)
        cust = plsc.load_gather(cidx_v, dims)
        blk = lax.shift_right_logical(cust, 7)
        mask = (blk >= blk0) & (blk < blk0 + nblk_w)
        pos = cnt + plsc.cumsum(mask.astype(jnp.int32)) - 1
        plsc.store_scatter(ent_ex, [pos], e, mask=mask)
        plsc.store_scatter(ent_cust, [pos], cust, mask=mask)
        return cnt + plsc.all_reduce_population_count(mask)[0]

    n_ent = lax.fori_loop(0, BATCH // LANES, scan_step, jnp.int32(0))

    def zero_step(i, c):
        counts_s[i] = 0
        return c
    lax.fori_loop(0, BLK_PER_W + 2, zero_step, 0)

    def bucket_step(i, c):
        cust = _scalar(ent_cust, i)
        ex = _scalar(ent_ex, i)
        loc = lax.shift_right_logical(cust, 7) - blk0
        j = jnp.bitwise_and(cust, 127)
        p = counts_s[loc]

        @pl.when(p < BUCKET_CAP)
        def _():
            slot = loc * BUCKET_CAP + p
            plsc.store_scatter(buckets_ex, [jnp.full((LANES,), slot,
                                                     jnp.int32)],
                               jnp.full((LANES,), ex, jnp.int32),
                               mask=lax.iota(jnp.int32, LANES) == 0)
            plsc.store_scatter(buckets_j, [jnp.full((LANES,), slot,
                                                    jnp.int32)],
                               jnp.full((LANES,), j, jnp.int32),
                               mask=lax.iota(jnp.int32, LANES) == 0)
            counts_s[loc] = p + 1

        @pl.when(p >= BUCKET_CAP)
        def _():
            q = counts_s[BLK_PER_W]
            plsc.store_scatter(ovf_ex, [jnp.full((LANES,), q, jnp.int32)],
                               jnp.full((LANES,), ex, jnp.int32),
                               mask=lax.iota(jnp.int32, LANES) == 0)
            plsc.store_scatter(ovf_cust, [jnp.full((LANES,), q, jnp.int32)],
                               jnp.full((LANES,), cust, jnp.int32),
                               mask=lax.iota(jnp.int32, LANES) == 0)
            counts_s[BLK_PER_W] = q + 1
        return c

    lax.fori_loop(0, n_ent, bucket_step, 0)

    def start_fetch(b, buf, sem):
        off = pl.multiple_of((blk0 + b) * 128, 128)
        pltpu.async_copy(ctab_t.at[:, pl.ds(off, 128)], buf, sem)

    def wait_fetch(b, buf, sem):
        pltpu.make_async_copy(ctab_t.at[:, pl.ds(0, 128)], buf, sem).wait()

    def process_block(b, buf):
        nloc = jnp.minimum(counts_s[b], BUCKET_CAP)
        blk = blk0 + b

        def one(k, c):
            ex = _scalar(buckets_ex, b * BUCKET_CAP + k)
            j = _scalar(buckets_j, b * BUCKET_CAP + k)
            sc = counts_s[SC_CELL]
            ss = jnp.bitwise_and(sc, 7)

            @pl.when(sc >= 8)
            def _():
                pltpu.make_async_copy(stage.at[pl.ds(0, 128)],
                                      cgat_hbm.at[pl.ds(0, 128)],
                                      osem).wait()
            _extract_col(buf, j, stage, ss)
            pltpu.async_copy(
                stage.at[pl.ds(ss * 128, 128)],
                cgat_hbm.at[pl.ds(pl.multiple_of(ex * SLOT, 128), 128)],
                osem)
            counts_s[SC_CELL] = sc + 1
            return c

        lax.fori_loop(0, nloc, one, 0)

    def wanted(b):
        return (b < nblk_w) & (counts_s[b] > 0)

    @pl.when(wanted(0))
    def _():
        start_fetch(0, chunk_a, csem_a)

    def sweep2(t, carry):
        b0 = 2 * t
        b1 = 2 * t + 1

        @pl.when(wanted(b1))
        def _():
            start_fetch(b1, chunk_b, csem_b)

        @pl.when(wanted(b0))
        def _():
            wait_fetch(b0, chunk_a, csem_a)
            process_block(b0, chunk_a)

        @pl.when(wanted(b0 + 2))
        def _():
            start_fetch(b0 + 2, chunk_a, csem_a)

        @pl.when(wanted(b1))
        def _():
            wait_fetch(b1, chunk_b, csem_b)
            process_block(b1, chunk_b)
        return carry

    lax.fori_loop(0, (BLK_PER_W + 1) // 2, sweep2, 0)

    def ovf_step(i, c):
        cust = _scalar(ovf_cust, i)
        ex = _scalar(ovf_ex, i)
        blk = lax.shift_right_logical(cust, 7)
        j = jnp.bitwise_and(cust, 127)
        sc = counts_s[SC_CELL]
        ss = jnp.bitwise_and(sc, 7)

        @pl.when(sc >= 8)
        def _():
            pltpu.make_async_copy(stage.at[pl.ds(0, 128)],
                                  cgat_hbm.at[pl.ds(0, 128)], osem).wait()

        off = pl.multiple_of(blk * 128, 128)
        pltpu.sync_copy(ctab_t.at[:, pl.ds(off, 128)], chunk_a)
        _extract_col(chunk_a, j, stage, ss)
        pltpu.async_copy(
            stage.at[pl.ds(ss * 128, 128)],
            cgat_hbm.at[pl.ds(pl.multiple_of(ex * SLOT, 128), 128)], osem)
        counts_s[SC_CELL] = sc + 1
        return c

    lax.fori_loop(0, counts_s[BLK_PER_W], ovf_step, 0)

    n_out = jnp.minimum(counts_s[SC_CELL], 8)

    def drain_step(i, c):
        @pl.when(i < n_out)
        def _():
            pltpu.make_async_copy(stage.at[pl.ds(0, 128)],
                                  cgat_hbm.at[pl.ds(0, 128)], osem).wait()
        return c

    lax.fori_loop(0, 8, drain_step, 0)


def _idx_vec(idx_s, g):
    e = g * LANES + lax.iota(jnp.int32, LANES)
    return plsc.load_gather(idx_s, [lax.shift_right_logical(e, 7),
                                    jnp.bitwise_and(e, 127)])


def _fetch_grp(tab, idx_s, g, buf, sem):
    grp = lax.shift_right_logical(_idx_vec(idx_s, g), 3)
    for l in range(LANES):
        base8 = pl.multiple_of(grp[l] * 8, 8)
        pltpu.async_copy(tab.at[pl.ds(base8, 8), :], buf.at[l], sem)


def _wait_grp(tab, idx_s, g, buf, sem):
    grp = lax.shift_right_logical(_idx_vec(idx_s, g), 3)
    for l in range(LANES):
        base8 = pl.multiple_of(grp[l] * 8, 8)
        pltpu.make_async_copy(tab.at[pl.ds(base8, 8), :],
                              buf.at[l], sem).wait()


def _fetch_cust(cgat_hbm, wid, g, cbuf, sem):
    base = (wid * B_PER_W + g * LANES) * SLOT
    pltpu.async_copy(cgat_hbm.at[pl.ds(pl.multiple_of(base, SLOT),
                                       LANES * SLOT)], cbuf, sem)


def _dot_body(cidx_hbm, pidx_hbm, ptab, ctail_hbm, cgat_hbm, out_hbm,
              cidx_s, pidx_s, out_v, ctail_v, cbuf_a, cbuf_b,
              pbuf_a, pbuf_b, csem_a, csem_b, psem_a, psem_b):
    wid = lax.axis_index("s") * NUM_CORES + lax.axis_index("c")

    pltpu.sync_copy(cidx_hbm.at[wid], cidx_s)
    pltpu.sync_copy(pidx_hbm.at[wid], pidx_s)
    pltpu.sync_copy(ctail_hbm, ctail_v)

    lane = lax.iota(jnp.int32, LANES)

    def dot(g, cbuf, pbuf):
        cust = _idx_vec(cidx_s, g)
        tail = cust >= TAIL0
        trow = jnp.maximum(cust - TAIL0, 0)
        psub = jnp.bitwise_and(_idx_vec(pidx_s, g), 7)
        acc = jnp.zeros((LANES,), jnp.float32)
        for d in range(EMBED_DIM):
            dv = jnp.full((LANES,), d, jnp.int32)
            cv = plsc.load_gather(cbuf, [lane * SLOT + dv])
            tv = plsc.load_gather(ctail_v, [trow, dv])
            cv = jnp.where(tail, tv, cv)
            pv = plsc.load_gather(pbuf, [lane, psub, dv])
            acc = acc + cv * pv
        plsc.store_scatter(out_v, [g * LANES + lane], acc)

    def pair_body(t, carry):
        g0 = 2 * t
        g1 = 2 * t + 1
        _fetch_cust(cgat_hbm, wid, g0, cbuf_a, csem_a)
        _fetch_grp(ptab, pidx_s, g0, pbuf_a, psem_a)
        _fetch_cust(cgat_hbm, wid, g1, cbuf_b, csem_b)
        _fetch_grp(ptab, pidx_s, g1, pbuf_b, psem_b)
        pltpu.make_async_copy(cgat_hbm.at[pl.ds(0, LANES * SLOT)], cbuf_a,
                              csem_a).wait()
        _wait_grp(ptab, pidx_s, g0, pbuf_a, psem_a)
        dot(g0, cbuf_a, pbuf_a)
        pltpu.make_async_copy(cgat_hbm.at[pl.ds(0, LANES * SLOT)], cbuf_b,
                              csem_b).wait()
        _wait_grp(ptab, pidx_s, g1, pbuf_b, psem_b)
        dot(g1, cbuf_b, pbuf_b)
        return carry

    lax.fori_loop(0, GROUPS // 2, pair_body, 0)

    pltpu.sync_copy(out_v, out_hbm.at[pl.ds(wid * OUT_STRIDE, OUT_STRIDE)])


@jax.jit
def _run(customer, product, customer_table, product_table):
    mesh = plsc.VectorSubcoreMesh(core_axis_name="c", subcore_axis_name="s",
                                  num_cores=NUM_CORES,
                                  num_subcores=NUM_SUBCORES)
    cidx = jnp.pad(customer.reshape(NUM_WORKERS, 4, 128),
                   ((0, 0), (0, 4), (0, 0)))
    pidx = jnp.pad(product.reshape(NUM_WORKERS, 4, 128),
                   ((0, 0), (0, 4), (0, 0)))

    cgat = pl.kernel(
        _gather_body,
        out_type=jax.ShapeDtypeStruct((BATCH * SLOT,), jnp.float32),
        mesh=mesh,
        compiler_params=pltpu.CompilerParams(needs_layout_passes=False),
        scratch_types=[
            pltpu.VMEM((NUM_WORKERS, 8, 128), jnp.int32),
            pltpu.VMEM((BATCH,), jnp.int32),
            pltpu.VMEM((BATCH,), jnp.int32),
            pltpu.VMEM(((BLK_PER_W + 1) * BUCKET_CAP,), jnp.int32),
            pltpu.VMEM(((BLK_PER_W + 1) * BUCKET_CAP,), jnp.int32),
            pltpu.VMEM((BATCH,), jnp.int32),
            pltpu.VMEM((BATCH,), jnp.int32),
            pltpu.VMEM((EMBED_DIM, 128), jnp.float32),
            pltpu.VMEM((EMBED_DIM, 128), jnp.float32),
            pltpu.VMEM((8 * 128,), jnp.float32),
            pltpu.SMEM((BLK_PER_W + 2,), jnp.int32),
            pltpu.SemaphoreType.DMA,
            pltpu.SemaphoreType.DMA,
            pltpu.SemaphoreType.DMA,
        ],
    )(cidx, customer_table.T)

    padded = pl.kernel(
        _dot_body,
        out_type=jax.ShapeDtypeStruct((NUM_WORKERS * OUT_STRIDE,),
                                      jnp.float32),
        mesh=mesh,
        compiler_params=pltpu.CompilerParams(needs_layout_passes=False),
        scratch_types=[
            pltpu.VMEM((8, 128), jnp.int32),
            pltpu.VMEM((8, 128), jnp.int32),
            pltpu.VMEM((OUT_STRIDE,), jnp.float32),
            pltpu.VMEM((EMBED_DIM, EMBED_DIM), jnp.float32),
            pltpu.VMEM((LANES * SLOT,), jnp.float32),
            pltpu.VMEM((LANES * SLOT,), jnp.float32),
            pltpu.VMEM((LANES, 8, EMBED_DIM), jnp.float32),
            pltpu.VMEM((LANES, 8, EMBED_DIM), jnp.float32),
            pltpu.SemaphoreType.DMA,
            pltpu.SemaphoreType.DMA,
            pltpu.SemaphoreType.DMA,
            pltpu.SemaphoreType.DMA,
        ],
    )(cidx, pidx, product_table, customer_table[TAIL0:], cgat)
    return padded.reshape(NUM_WORKERS, OUT_STRIDE)[:, :B_PER_W].reshape(-1)


def kernel(customer, product, customer_table, product_table):
    return _run(customer, product, customer_table, product_table)


# speedup vs baseline: 21.1809x; 1.2071x over previous
import jax
import jax.numpy as jnp
from jax import lax
from jax.experimental import pallas as pl
from jax.experimental.pallas import tpu as pltpu
from jax.experimental.pallas import tpu_sc as plsc

NUM_CORES = 2
NUM_SUBCORES = 16
LANES = 16
NUM_WORKERS = NUM_CORES * NUM_SUBCORES

NUM_CUSTOMERS = 1000000
NUM_PRODUCTS = 100000
BATCH = 16384
EMBED_DIM = 64
B_PER_W = BATCH // NUM_WORKERS
GROUPS = B_PER_W // LANES
OUT_STRIDE = 2 * B_PER_W
SLOT = 128

NBLK = NUM_CUSTOMERS // 128
TAIL0 = NBLK * 128
BLK_PER_W = (NBLK + NUM_WORKERS - 1) // NUM_WORKERS
BUCKET_CAP = 8
SC_CELL = BLK_PER_W + 1


def _evec(i):
    e = i * LANES + lax.iota(jnp.int32, LANES)
    return e, [lax.shift_right_logical(e, 9),
               jnp.bitwise_and(lax.shift_right_logical(e, 7), 3),
               jnp.bitwise_and(e, 127)]


def _scalar(ref, i):
    v = plsc.load_gather(ref, [jnp.full((LANES,), i, jnp.int32)])
    return v[0]


def _extract_col(chunk, j, stage, slot):
    jv = jnp.full((LANES,), j, jnp.int32)
    lane = lax.iota(jnp.int32, LANES)
    for q in range(4):
        rows = q * LANES + lane
        v = plsc.load_gather(chunk, [rows, jv])
        plsc.store_scatter(stage, [slot * 128 + q * LANES + lane], v)


def _gather_body(cidx_hbm, ctab_t, cgat_hbm,
                 cidx_v, ent_ex, ent_cust, buckets_ex, buckets_j,
                 chunk_a, chunk_b, chunk_c, chunk_d, stage,
                 counts_s, csem_a, csem_b, csem_c, csem_d, osem):
    wid = lax.axis_index("s") * NUM_CORES + lax.axis_index("c")
    blk0 = wid * BLK_PER_W
    nblk_w = jnp.minimum(jnp.int32(NBLK) - blk0, BLK_PER_W)
    nblk_w = jnp.maximum(nblk_w, 0)

    pltpu.sync_copy(cidx_hbm, cidx_v)

    def scan_step(i, cnt):
        e, dims = _evec(i)
        cust = plsc.load_gather(cidx_v, dims)
        blk = lax.shift_right_logical(cust, 7)
        mask = (blk >= blk0) & (blk < blk0 + nblk_w)
        pos = cnt + plsc.cumsum(mask.astype(jnp.int32)) - 1
        plsc.store_scatter(ent_ex, [pos], e, mask=mask)
        plsc.store_scatter(ent_cust, [pos], cust, mask=mask)
        return cnt + plsc.all_reduce_population_count(mask)[0]

    n_ent = lax.fori_loop(0, BATCH // LANES, scan_step, jnp.int32(0))

    def zero_step(i, c):
        counts_s[i] = 0
        return c
    lax.fori_loop(0, BLK_PER_W + 2, zero_step, 0)

    def bucket_step(i, c):
        cust = _scalar(ent_cust, i)
        ex = _scalar(ent_ex, i)
        loc = lax.shift_right_logical(cust, 7) - blk0
        j = jnp.bitwise_and(cust, 127)
        p = counts_s[loc]

        @pl.when(p < BUCKET_CAP)
        def _():
            slot = loc * BUCKET_CAP + p
            plsc.store_scatter(buckets_ex, [jnp.full((LANES,), slot,
                                                     jnp.int32)],
                               jnp.full((LANES,), ex, jnp.int32),
                               mask=lax.iota(jnp.int32, LANES) == 0)
            plsc.store_scatter(buckets_j, [jnp.full((LANES,), slot,
                                                    jnp.int32)],
                               jnp.full((LANES,), j, jnp.int32),
                               mask=lax.iota(jnp.int32, LANES) == 0)
            counts_s[loc] = p + 1

        @pl.when(p >= BUCKET_CAP)
        def _():
            q = counts_s[BLK_PER_W]
            plsc.store_scatter(ent_ex, [jnp.full((LANES,), q, jnp.int32)],
                               jnp.full((LANES,), ex, jnp.int32),
                               mask=lax.iota(jnp.int32, LANES) == 0)
            plsc.store_scatter(ent_cust, [jnp.full((LANES,), q, jnp.int32)],
                               jnp.full((LANES,), cust, jnp.int32),
                               mask=lax.iota(jnp.int32, LANES) == 0)
            counts_s[BLK_PER_W] = q + 1
        return c

    lax.fori_loop(0, n_ent, bucket_step, 0)

    def start_fetch(b, buf, sem):
        off = pl.multiple_of((blk0 + b) * 128, 128)
        pltpu.async_copy(ctab_t.at[:, pl.ds(off, 128)], buf, sem)

    def wait_fetch(b, buf, sem):
        pltpu.make_async_copy(ctab_t.at[:, pl.ds(0, 128)], buf, sem).wait()

    def process_block(b, buf):
        nloc = jnp.minimum(counts_s[b], BUCKET_CAP)
        blk = blk0 + b

        def one(k, c):
            ex = _scalar(buckets_ex, b * BUCKET_CAP + k)
            j = _scalar(buckets_j, b * BUCKET_CAP + k)
            sc = counts_s[SC_CELL]
            ss = jnp.bitwise_and(sc, 7)

            @pl.when(sc >= 8)
            def _():
                pltpu.make_async_copy(stage.at[pl.ds(0, 128)],
                                      cgat_hbm.at[pl.ds(0, 128)],
                                      osem).wait()
            _extract_col(buf, j, stage, ss)
            pltpu.async_copy(
                stage.at[pl.ds(ss * 128, 128)],
                cgat_hbm.at[pl.ds(pl.multiple_of(ex * SLOT, 128), 128)],
                osem)
            counts_s[SC_CELL] = sc + 1
            return c

        lax.fori_loop(0, nloc, one, 0)

    def wanted(b):
        return (b < nblk_w) & (counts_s[jnp.minimum(b, BLK_PER_W)] > 0)

    ring = ((chunk_a, csem_a), (chunk_b, csem_b),
            (chunk_c, csem_c), (chunk_d, csem_d))

    for k in range(4):
        @pl.when(wanted(k))
        def _(k=k):
            start_fetch(k, *ring[k])

    def sweep4(t, carry):
        for k in range(4):
            b = 4 * t + k

            @pl.when(wanted(b))
            def _(b=b, k=k):
                wait_fetch(b, *ring[k])
                process_block(b, ring[k][0])

            @pl.when(wanted(b + 4))
            def _(b=b, k=k):
                start_fetch(b + 4, *ring[k])
        return carry

    lax.fori_loop(0, (BLK_PER_W + 3) // 4, sweep4, 0)

    def ovf_step(i, c):
        cust = _scalar(ent_cust, i)
        ex = _scalar(ent_ex, i)
        blk = lax.shift_right_logical(cust, 7)
        j = jnp.bitwise_and(cust, 127)
        sc = counts_s[SC_CELL]
        ss = jnp.bitwise_and(sc, 7)

        @pl.when(sc >= 8)
        def _():
            pltpu.make_async_copy(stage.at[pl.ds(0, 128)],
                                  cgat_hbm.at[pl.ds(0, 128)], osem).wait()

        off = pl.multiple_of(blk * 128, 128)
        pltpu.sync_copy(ctab_t.at[:, pl.ds(off, 128)], chunk_a)
        _extract_col(chunk_a, j, stage, ss)
        pltpu.async_copy(
            stage.at[pl.ds(ss * 128, 128)],
            cgat_hbm.at[pl.ds(pl.multiple_of(ex * SLOT, 128), 128)], osem)
        counts_s[SC_CELL] = sc + 1
        return c

    lax.fori_loop(0, counts_s[BLK_PER_W], ovf_step, 0)

    n_out = jnp.minimum(counts_s[SC_CELL], 8)

    def drain_step(i, c):
        @pl.when(i < n_out)
        def _():
            pltpu.make_async_copy(stage.at[pl.ds(0, 128)],
                                  cgat_hbm.at[pl.ds(0, 128)], osem).wait()
        return c

    lax.fori_loop(0, 8, drain_step, 0)


def _idx_vec(idx_s, g):
    e = g * LANES + lax.iota(jnp.int32, LANES)
    return plsc.load_gather(idx_s, [lax.shift_right_logical(e, 7),
                                    jnp.bitwise_and(e, 127)])


def _fetch_grp(tab, idx_s, g, buf, sem):
    grp = lax.shift_right_logical(_idx_vec(idx_s, g), 3)
    for l in range(LANES):
        base8 = pl.multiple_of(grp[l] * 8, 8)
        pltpu.async_copy(tab.at[pl.ds(base8, 8), :], buf.at[l], sem)


def _wait_grp(tab, idx_s, g, buf, sem):
    grp = lax.shift_right_logical(_idx_vec(idx_s, g), 3)
    for l in range(LANES):
        base8 = pl.multiple_of(grp[l] * 8, 8)
        pltpu.make_async_copy(tab.at[pl.ds(base8, 8), :],
                              buf.at[l], sem).wait()


def _fetch_cust(cgat_hbm, wid, g, cbuf, sem):
    base = (wid * B_PER_W + g * LANES) * SLOT
    pltpu.async_copy(cgat_hbm.at[pl.ds(pl.multiple_of(base, SLOT),
                                       LANES * SLOT)], cbuf, sem)


def _dot_body(cidx_hbm, pidx_hbm, ptab, ctail_hbm, cgat_hbm, out_hbm,
              cidx_s, pidx_s, out_v, ctail_v, cbuf_a, cbuf_b,
              pbuf_a, pbuf_b, csem_a, csem_b, psem_a, psem_b):
    wid = lax.axis_index("s") * NUM_CORES + lax.axis_index("c")

    pltpu.sync_copy(cidx_hbm.at[wid], cidx_s)
    pltpu.sync_copy(pidx_hbm.at[wid], pidx_s)
    pltpu.sync_copy(ctail_hbm, ctail_v)

    lane = lax.iota(jnp.int32, LANES)

    def dot(g, cbuf, pbuf):
        cust = _idx_vec(cidx_s, g)
        tail = cust >= TAIL0
        trow = jnp.maximum(cust - TAIL0, 0)
        psub = jnp.bitwise_and(_idx_vec(pidx_s, g), 7)
        acc = jnp.zeros((LANES,), jnp.float32)
        for d in range(EMBED_DIM):
            dv = jnp.full((LANES,), d, jnp.int32)
            cv = plsc.load_gather(cbuf, [lane * SLOT + dv])
            tv = plsc.load_gather(ctail_v, [trow, dv])
            cv = jnp.where(tail, tv, cv)
            pv = plsc.load_gather(pbuf, [lane, psub, dv])
            acc = acc + cv * pv
        plsc.store_scatter(out_v, [g * LANES + lane], acc)

    def pair_body(t, carry):
        g0 = 2 * t
        g1 = 2 * t + 1
        _fetch_cust(cgat_hbm, wid, g0, cbuf_a, csem_a)
        _fetch_grp(ptab, pidx_s, g0, pbuf_a, psem_a)
        _fetch_cust(cgat_hbm, wid, g1, cbuf_b, csem_b)
        _fetch_grp(ptab, pidx_s, g1, pbuf_b, psem_b)
        pltpu.make_async_copy(cgat_hbm.at[pl.ds(0, LANES * SLOT)], cbuf_a,
                              csem_a).wait()
        _wait_grp(ptab, pidx_s, g0, pbuf_a, psem_a)
        dot(g0, cbuf_a, pbuf_a)
        pltpu.make_async_copy(cgat_hbm.at[pl.ds(0, LANES * SLOT)], cbuf_b,
                              csem_b).wait()
        _wait_grp(ptab, pidx_s, g1, pbuf_b, psem_b)
        dot(g1, cbuf_b, pbuf_b)
        return carry

    lax.fori_loop(0, GROUPS // 2, pair_body, 0)

    pltpu.sync_copy(out_v, out_hbm.at[pl.ds(wid * OUT_STRIDE, OUT_STRIDE)])


@jax.jit
def _run(customer, product, customer_table, product_table):
    mesh = plsc.VectorSubcoreMesh(core_axis_name="c", subcore_axis_name="s",
                                  num_cores=NUM_CORES,
                                  num_subcores=NUM_SUBCORES)
    cidx = jnp.pad(customer.reshape(NUM_WORKERS, 4, 128),
                   ((0, 0), (0, 4), (0, 0)))
    pidx = jnp.pad(product.reshape(NUM_WORKERS, 4, 128),
                   ((0, 0), (0, 4), (0, 0)))

    cgat = pl.kernel(
        _gather_body,
        out_type=jax.ShapeDtypeStruct((BATCH * SLOT,), jnp.float32),
        mesh=mesh,
        compiler_params=pltpu.CompilerParams(needs_layout_passes=False),
        scratch_types=[
            pltpu.VMEM((NUM_WORKERS, 8, 128), jnp.int32),
            pltpu.VMEM((BATCH,), jnp.int32),
            pltpu.VMEM((BATCH,), jnp.int32),
            pltpu.VMEM(((BLK_PER_W + 1) * BUCKET_CAP,), jnp.int32),
            pltpu.VMEM(((BLK_PER_W + 1) * BUCKET_CAP,), jnp.int32),
            pltpu.VMEM((EMBED_DIM, 128), jnp.float32),
            pltpu.VMEM((EMBED_DIM, 128), jnp.float32),
            pltpu.VMEM((EMBED_DIM, 128), jnp.float32),
            pltpu.VMEM((EMBED_DIM, 128), jnp.float32),
            pltpu.VMEM((8 * 128,), jnp.float32),
            pltpu.SMEM((BLK_PER_W + 2,), jnp.int32),
            pltpu.SemaphoreType.DMA,
            pltpu.SemaphoreType.DMA,
            pltpu.SemaphoreType.DMA,
            pltpu.SemaphoreType.DMA,
            pltpu.SemaphoreType.DMA,
        ],
    )(cidx, customer_table.T)

    padded = pl.kernel(
        _dot_body,
        out_type=jax.ShapeDtypeStruct((NUM_WORKERS * OUT_STRIDE,),
                                      jnp.float32),
        mesh=mesh,
        compiler_params=pltpu.CompilerParams(needs_layout_passes=False),
        scratch_types=[
            pltpu.VMEM((8, 128), jnp.int32),
            pltpu.VMEM((8, 128), jnp.int32),
            pltpu.VMEM((OUT_STRIDE,), jnp.float32),
            pltpu.VMEM((EMBED_DIM, EMBED_DIM), jnp.float32),
            pltpu.VMEM((LANES * SLOT,), jnp.float32),
            pltpu.VMEM((LANES * SLOT,), jnp.float32),
            pltpu.VMEM((LANES, 8, EMBED_DIM), jnp.float32),
            pltpu.VMEM((LANES, 8, EMBED_DIM), jnp.float32),
            pltpu.SemaphoreType.DMA,
            pltpu.SemaphoreType.DMA,
            pltpu.SemaphoreType.DMA,
            pltpu.SemaphoreType.DMA,
        ],
    )(cidx, pidx, product_table, customer_table[TAIL0:], cgat)
    return padded.reshape(NUM_WORKERS, OUT_STRIDE)[:, :B_PER_W].reshape(-1)


def kernel(customer, product, customer_table, product_table):
    return _run(customer, product, customer_table, product_table)
